# baseline (speedup 1.0000x reference)
import functools

import jax
import jax.numpy as jnp
from jax import lax
from jax.experimental import pallas as pl
from jax.experimental.pallas import tpu as pltpu
from jax.experimental.pallas import tpu_sc as plsc

NCLS = 22
CHUNK = 256


def _rup8(x):
    return ((x + 7) // 8) * 8


def _div20(x):
    return ((x >> 2) * 13108) >> 16


def _div24(x):
    return ((x >> 3) * 21846) >> 16


def _div56(x):
    return ((x >> 3) * 9363) >> 16


def _sc_gather(dist, angle, index_t, qvec2d, pre_p, wtab, h, w, L):
    info = plsc.get_sparse_core_info()
    NC, NS, LN = info.num_cores, info.num_subcores, info.num_lanes
    NW = NC * NS
    hb = h // NW
    n_chunks = hb // CHUNK
    cw = CHUNK * w
    pd_ = _rup8(L)
    pt_ = _rup8(w)
    mesh = plsc.VectorSubcoreMesh(core_axis_name="c", subcore_axis_name="s")

    @functools.partial(
        pl.kernel,
        out_type=(
            jax.ShapeDtypeStruct((h, pt_), jnp.float32),
            jax.ShapeDtypeStruct((h, pt_), jnp.float32),
            jax.ShapeDtypeStruct((NW, LN), jnp.float32),
            jax.ShapeDtypeStruct((NW, LN), jnp.float32),
        ),
        mesh=mesh,
        compiler_params=pltpu.CompilerParams(needs_layout_passes=False,
                                             use_tc_tiling_on_sc=False),
        scratch_types=(
            pltpu.VMEM((2, 128), jnp.int32),
            pltpu.VMEM((2, 128), jnp.int32),
            pltpu.VMEM((4, 128), jnp.int32),
            pltpu.VMEM((2 * CHUNK, L), jnp.float32),
            pltpu.VMEM((2 * CHUNK, L), jnp.float32),
            pltpu.VMEM((2 * CHUNK, w), jnp.int32),
            pltpu.VMEM((CHUNK,), jnp.int32),
            pltpu.VMEM((hb, pt_), jnp.float32),
            pltpu.VMEM((hb, pt_), jnp.float32),
            pltpu.VMEM((LN,), jnp.float32),
            pltpu.VMEM((LN,), jnp.float32),
            pltpu.SemaphoreType.DMA,
        ),
    )
    def k(dist_hbm, angle_hbm, indext_hbm, qvec_hbm, prep_hbm, wtab_hbm,
          dist_g, angle_g, mins, maxs,
          idx_s, idx1_s, wt_s, dist_rows, angle_rows, idxt_rows, pre_v,
          dist_o, angle_o, min_v, max_v, sem):
        wid = lax.axis_index("s") * NC + lax.axis_index("c")
        inf = jnp.full((LN,), jnp.inf, dtype=jnp.float32)
        zero = jnp.zeros((LN,), dtype=jnp.float32)
        iota = lax.broadcasted_iota(jnp.int32, (LN,), 0)
        vmin, vmax = inf, -inf
        for ch in range(n_chunks):
            base_row = wid * hb + ch * CHUNK
            sA = base_row // 128
            pltpu.sync_copy(qvec_hbm.at[pl.ds(sA, 2)], idx_s)
            pltpu.sync_copy(wtab_hbm.at[pl.ds(2 * sA, 4)], wt_s)
            pltpu.sync_copy(prep_hbm.at[pl.ds(base_row, CHUNK)], pre_v)
            for r2 in range(2):
                for kk in range(128 // LN):
                    idx1_s[r2, pl.ds(LN * kk, LN)] = (
                        idx_s[r2, pl.ds(LN * kk, LN)] + 1)
            cps = []
            for sub in range(2):
                cps.append(pltpu.async_copy(
                    dist_hbm.at[idx_s.at[sub]],
                    dist_rows.at[pl.ds((2 * sub) * 128, 128)], sem))
                cps.append(pltpu.async_copy(
                    dist_hbm.at[idx1_s.at[sub]],
                    dist_rows.at[pl.ds((2 * sub + 1) * 128, 128)], sem))
                cps.append(pltpu.async_copy(
                    angle_hbm.at[idx_s.at[sub]],
                    angle_rows.at[pl.ds((2 * sub) * 128, 128)], sem))
                cps.append(pltpu.async_copy(
                    angle_hbm.at[idx1_s.at[sub]],
                    angle_rows.at[pl.ds((2 * sub + 1) * 128, 128)], sem))
            for kk in range(4):
                cps.append(pltpu.async_copy(
                    indext_hbm.at[wt_s.at[kk]],
                    idxt_rows.at[pl.ds(kk * 128, 128)], sem))
            for cp in cps:
                cp.wait()

            def body(g, carry, _ch=ch):
                mn, mx = carry
                e = g * LN + iota
                p = _div20(e)
                j = e - 20 * p
                prev = plsc.load_gather(pre_v, [p])
                ob_d = prev & 63
                ob_t = prev >> 6
                sub = p >> 7
                pdv = p & 127
                ot = ob_t + j
                hit = (ot >= w).astype(jnp.int32)
                wt = 128 * pt_ * (2 * sub + hit) + w * pdv + ot - w * hit
                rowt = _div24(wt)
                colt = wt - pt_ * rowt
                c = plsc.load_gather(idxt_rows, [rowt, colt])
                inv = c >= L
                od = ob_d + c
                hid = (od >= L).astype(jnp.int32)
                wd = 128 * pd_ * (2 * sub + hid) + L * pdv + od - L * hid
                rowd = _div56(wd)
                cold = wd - pd_ * rowd
                vd = plsc.load_gather(dist_rows, [rowd, cold])
                va = plsc.load_gather(angle_rows, [rowd, cold])
                vd = jnp.where(inv, zero, vd)
                va = jnp.where(inv, zero, va)
                orow = _ch * CHUNK + p
                plsc.store_scatter(dist_o, [orow, j], vd)
                plsc.store_scatter(angle_o, [orow, j], va)
                return jnp.minimum(mn, vd), jnp.maximum(mx, vd)

            vmin, vmax = lax.fori_loop(0, cw // LN, body, (vmin, vmax))

        min_v[...] = vmin
        max_v[...] = vmax
        pltpu.sync_copy(dist_o, dist_g.at[pl.ds(wid * hb, hb)])
        pltpu.sync_copy(angle_o, angle_g.at[pl.ds(wid * hb, hb)])
        pltpu.sync_copy(min_v, mins.at[wid])
        pltpu.sync_copy(max_v, maxs.at[wid])

    return k(dist, angle, index_t, qvec2d, pre_p, wtab)


def _tc_assemble(idx_t, dist_g, angle_g, mins, maxs, h, w, interpret=False):
    out_w = NCLS * w + 2 * w
    BH = 2048
    grid = (h // BH,)
    pt_ = dist_g.shape[1]

    def body(idx_ref, dist_ref, angle_ref, mins_ref, maxs_ref, out_ref):
        gmin = jnp.min(mins_ref[...])
        gmax = jnp.max(maxs_ref[...])
        scale = 1.0 / (gmax - gmin)
        idxf = idx_ref[...].astype(jnp.bfloat16)
        qj = lax.broadcasted_iota(jnp.int32, (w, NCLS * w), 1) // NCLS
        jj = lax.broadcasted_iota(jnp.int32, (w, NCLS * w), 0)
        sel = (qj == jj).astype(jnp.bfloat16)
        rep = jnp.dot(idxf, sel, preferred_element_type=jnp.float32)
        cls = (lax.broadcasted_iota(jnp.int32, (BH, NCLS * w), 1)
               % NCLS).astype(jnp.float32)
        out_ref[:, :NCLS * w] = (rep == cls).astype(jnp.float32)
        out_ref[:, NCLS * w:NCLS * w + w] = (
            dist_ref[...][:, :w] - gmin) * scale
        out_ref[:, NCLS * w + w:] = angle_ref[...][:, :w]

    return pl.pallas_call(
        body,
        grid=grid,
        in_specs=[
            pl.BlockSpec((BH, w), lambda i: (i, 0)),
            pl.BlockSpec((BH, pt_), lambda i: (i, 0)),
            pl.BlockSpec((BH, pt_), lambda i: (i, 0)),
            pl.BlockSpec(mins.shape, lambda i: (0, 0)),
            pl.BlockSpec(maxs.shape, lambda i: (0, 0)),
        ],
        out_specs=pl.BlockSpec((BH, out_w), lambda i: (i, 0)),
        out_shape=jax.ShapeDtypeStruct((h, out_w), jnp.float32),
        interpret=interpret,
    )(idx_t, dist_g, angle_g, mins, maxs)


def _precompute(index_h, h, w, L):
    pitch_d = _rup8(L)
    pitch_t = _rup8(w)
    r = index_h.astype(jnp.int32)
    w0 = r * pitch_d
    q = w0 // L
    ob_d = w0 - q * L
    ar = jnp.arange(h, dtype=jnp.int32)
    w0t = ar * pitch_t
    qt = w0t // w
    ob_t = w0t - qt * w
    pre_p = ob_d | (ob_t << 6)
    qvec2d = q.reshape(h // 128, 128)
    wtab = jnp.stack([qt.reshape(-1, 128), qt.reshape(-1, 128) + 1],
                     axis=1).reshape(-1, 128)
    return qvec2d, pre_p, wtab


def kernel(dist, angle, idx_t, index_t, index_h):
    N, L = dist.shape
    h, w = idx_t.shape
    qvec2d, pre_p, wtab = _precompute(index_h, h, w, L)
    dist_g, angle_g, mins, maxs = _sc_gather(dist, angle, index_t, qvec2d,
                                             pre_p, wtab, h, w, L)
    return _tc_assemble(idx_t, dist_g, angle_g, mins, maxs, h, w)


# speedup vs baseline: 1.6482x; 1.6482x over previous
import functools

import jax
import jax.numpy as jnp
from jax import lax
from jax.experimental import pallas as pl
from jax.experimental.pallas import tpu as pltpu
from jax.experimental.pallas import tpu_sc as plsc

NCLS = 22
CHUNK = 128
PITCH = 128


def _rup8(x):
    return ((x + 7) // 8) * 8


def _div20(x):
    return ((x >> 2) * 13108) >> 16


def _tc_relayout(tt):
    d, n = tt.shape
    BC = 2048
    nb = (n + BC - 1) // BC

    def body(in_ref, out_ref):
        out_ref[:, :d] = in_ref[...].T

    return pl.pallas_call(
        body,
        grid=(nb,),
        in_specs=[pl.BlockSpec((d, BC), lambda i: (0, i))],
        out_specs=pl.BlockSpec((BC, PITCH), lambda i: (i, 0)),
        out_shape=jax.ShapeDtypeStruct((nb * BC, PITCH), tt.dtype),
    )(tt)


def _sc_gather(dist_r, angle_r, idxt_r, qvec2d, h, w, L):
    info = plsc.get_sparse_core_info()
    NC, NS, LN = info.num_cores, info.num_subcores, info.num_lanes
    NW = NC * NS
    hb = h // NW
    n_chunks = hb // CHUNK
    cw = CHUNK * w
    pt_ = _rup8(w)
    mesh = plsc.VectorSubcoreMesh(core_axis_name="c", subcore_axis_name="s")

    @functools.partial(
        pl.kernel,
        out_type=(
            jax.ShapeDtypeStruct((h, pt_), jnp.float32),
            jax.ShapeDtypeStruct((h, pt_), jnp.float32),
            jax.ShapeDtypeStruct((NW, LN), jnp.float32),
            jax.ShapeDtypeStruct((NW, LN), jnp.float32),
        ),
        mesh=mesh,
        compiler_params=pltpu.CompilerParams(needs_layout_passes=False,
                                             use_tc_tiling_on_sc=False),
        scratch_types=(
            pltpu.VMEM((1, 128), jnp.int32),
            pltpu.VMEM((CHUNK, PITCH), jnp.float32),
            pltpu.VMEM((CHUNK, PITCH), jnp.float32),
            pltpu.VMEM((CHUNK, PITCH), jnp.int32),
            pltpu.VMEM((hb, pt_), jnp.float32),
            pltpu.VMEM((hb, pt_), jnp.float32),
            pltpu.VMEM((LN,), jnp.float32),
            pltpu.VMEM((LN,), jnp.float32),
            pltpu.SemaphoreType.DMA,
        ),
    )
    def k(dist_hbm, angle_hbm, idxt_hbm, qvec_hbm,
          dist_g, angle_g, mins, maxs,
          idx_s, dist_rows, angle_rows, idxt_rows,
          dist_o, angle_o, min_v, max_v, sem):
        wid = lax.axis_index("s") * NC + lax.axis_index("c")
        inf = jnp.full((LN,), jnp.inf, dtype=jnp.float32)
        zero = jnp.zeros((LN,), dtype=jnp.float32)
        iota = lax.broadcasted_iota(jnp.int32, (LN,), 0)
        vmin, vmax = inf, -inf
        for ch in range(n_chunks):
            base_row = wid * hb + ch * CHUNK
            pltpu.sync_copy(qvec_hbm.at[pl.ds(base_row // 128, 1)], idx_s)
            cps = [
                pltpu.async_copy(dist_hbm.at[idx_s.at[0]], dist_rows, sem),
                pltpu.async_copy(angle_hbm.at[idx_s.at[0]], angle_rows, sem),
                pltpu.async_copy(idxt_hbm.at[pl.ds(base_row, CHUNK)],
                                 idxt_rows, sem),
            ]
            for cp in cps:
                cp.wait()

            def body(g, carry, _ch=ch):
                mn, mx = carry
                e = g * LN + iota
                p = _div20(e)
                j = e - 20 * p
                c = plsc.load_gather(idxt_rows, [p, j])
                inv = c >= L
                vd = plsc.load_gather(dist_rows, [p, c])
                va = plsc.load_gather(angle_rows, [p, c])
                vd = jnp.where(inv, zero, vd)
                va = jnp.where(inv, zero, va)
                orow = _ch * CHUNK + p
                plsc.store_scatter(dist_o, [orow, j], vd)
                plsc.store_scatter(angle_o, [orow, j], va)
                return jnp.minimum(mn, vd), jnp.maximum(mx, vd)

            vmin, vmax = lax.fori_loop(0, cw // LN, body, (vmin, vmax))

        min_v[...] = vmin
        max_v[...] = vmax
        pltpu.sync_copy(dist_o, dist_g.at[pl.ds(wid * hb, hb)])
        pltpu.sync_copy(angle_o, angle_g.at[pl.ds(wid * hb, hb)])
        pltpu.sync_copy(min_v, mins.at[wid])
        pltpu.sync_copy(max_v, maxs.at[wid])

    return k(dist_r, angle_r, idxt_r, qvec2d)


def _tc_assemble(idx_t, dist_g, angle_g, mins, maxs, h, w, interpret=False):
    out_w = NCLS * w + 2 * w
    BH = 2048
    grid = (h // BH,)
    pt_ = dist_g.shape[1]

    def body(idx_ref, dist_ref, angle_ref, mins_ref, maxs_ref, out_ref):
        gmin = jnp.min(mins_ref[...])
        gmax = jnp.max(maxs_ref[...])
        scale = 1.0 / (gmax - gmin)
        idxf = idx_ref[...].astype(jnp.bfloat16)
        qj = lax.broadcasted_iota(jnp.int32, (w, NCLS * w), 1) // NCLS
        jj = lax.broadcasted_iota(jnp.int32, (w, NCLS * w), 0)
        sel = (qj == jj).astype(jnp.bfloat16)
        rep = jnp.dot(idxf, sel, preferred_element_type=jnp.float32)
        cls = (lax.broadcasted_iota(jnp.int32, (BH, NCLS * w), 1)
               % NCLS).astype(jnp.float32)
        out_ref[:, :NCLS * w] = (rep == cls).astype(jnp.float32)
        out_ref[:, NCLS * w:NCLS * w + w] = (
            dist_ref[...][:, :w] - gmin) * scale
        out_ref[:, NCLS * w + w:] = angle_ref[...][:, :w]

    return pl.pallas_call(
        body,
        grid=grid,
        in_specs=[
            pl.BlockSpec((BH, w), lambda i: (i, 0)),
            pl.BlockSpec((BH, pt_), lambda i: (i, 0)),
            pl.BlockSpec((BH, pt_), lambda i: (i, 0)),
            pl.BlockSpec(mins.shape, lambda i: (0, 0)),
            pl.BlockSpec(maxs.shape, lambda i: (0, 0)),
        ],
        out_specs=pl.BlockSpec((BH, out_w), lambda i: (i, 0)),
        out_shape=jax.ShapeDtypeStruct((h, out_w), jnp.float32),
        interpret=interpret,
    )(idx_t, dist_g, angle_g, mins, maxs)


def kernel(dist, angle, idx_t, index_t, index_h):
    N, L = dist.shape
    h, w = idx_t.shape
    dist_r = _tc_relayout(dist.T)
    angle_r = _tc_relayout(angle.T)
    idxt_r = _tc_relayout(index_t.T)
    qvec2d = index_h.astype(jnp.int32).reshape(h // 128, 128)
    dist_g, angle_g, mins, maxs = _sc_gather(dist_r, angle_r, idxt_r,
                                             qvec2d, h, w, L)
    return _tc_assemble(idx_t, dist_g, angle_g, mins, maxs, h, w)


# speedup vs baseline: 1.8581x; 1.1274x over previous
import functools

import jax
import jax.numpy as jnp
from jax import lax
from jax.experimental import pallas as pl
from jax.experimental.pallas import tpu as pltpu
from jax.experimental.pallas import tpu_sc as plsc

NCLS = 22
CHUNK = 128
PITCH = 128


def _rup8(x):
    return ((x + 7) // 8) * 8


def _div20(x):
    return ((x >> 2) * 13108) >> 16


def _tc_relayout(tt):
    d, n = tt.shape
    BC = 2048
    nb = (n + BC - 1) // BC

    def body(in_ref, out_ref):
        out_ref[:, :d] = in_ref[...].T

    return pl.pallas_call(
        body,
        grid=(nb,),
        in_specs=[pl.BlockSpec((d, BC), lambda i: (0, i))],
        out_specs=pl.BlockSpec((BC, PITCH), lambda i: (i, 0)),
        out_shape=jax.ShapeDtypeStruct((nb * BC, PITCH), tt.dtype),
    )(tt)


def _sc_gather(dist_r, angle_r, idxt_r, qvec2d, h, w, L):
    info = plsc.get_sparse_core_info()
    NC, NS, LN = info.num_cores, info.num_subcores, info.num_lanes
    NW = NC * NS
    hb = h // NW
    n_chunks = hb // CHUNK
    cw = CHUNK * w
    pt_ = _rup8(w)
    mesh = plsc.VectorSubcoreMesh(core_axis_name="c", subcore_axis_name="s")

    @functools.partial(
        pl.kernel,
        out_type=(
            jax.ShapeDtypeStruct((h, pt_), jnp.float32),
            jax.ShapeDtypeStruct((h, pt_), jnp.float32),
            jax.ShapeDtypeStruct((NW, LN), jnp.float32),
            jax.ShapeDtypeStruct((NW, LN), jnp.float32),
        ),
        mesh=mesh,
        compiler_params=pltpu.CompilerParams(needs_layout_passes=False,
                                             use_tc_tiling_on_sc=False),
        scratch_types=(
            pltpu.VMEM((1, 128), jnp.int32),
            pltpu.VMEM((CHUNK, PITCH), jnp.float32),
            pltpu.VMEM((CHUNK, PITCH), jnp.float32),
            pltpu.VMEM((CHUNK, PITCH), jnp.int32),
            pltpu.VMEM((hb, pt_), jnp.float32),
            pltpu.VMEM((hb, pt_), jnp.float32),
            pltpu.VMEM((LN,), jnp.float32),
            pltpu.VMEM((LN,), jnp.float32),
            pltpu.SemaphoreType.DMA,
        ),
    )
    def k(dist_hbm, angle_hbm, idxt_hbm, qvec_hbm,
          dist_g, angle_g, mins, maxs,
          idx_s, dist_rows, angle_rows, idxt_rows,
          dist_o, angle_o, min_v, max_v, sem):
        wid = lax.axis_index("s") * NC + lax.axis_index("c")
        inf = jnp.full((LN,), jnp.inf, dtype=jnp.float32)
        zero = jnp.zeros((LN,), dtype=jnp.float32)
        iota = lax.broadcasted_iota(jnp.int32, (LN,), 0)
        vmin, vmax = inf, -inf
        for ch in range(n_chunks):
            base_row = wid * hb + ch * CHUNK
            pltpu.sync_copy(qvec_hbm.at[pl.ds(base_row // 128, 1)], idx_s)
            cps = [
                pltpu.async_copy(dist_hbm.at[idx_s.at[0]], dist_rows, sem),
                pltpu.async_copy(angle_hbm.at[idx_s.at[0]], angle_rows, sem),
                pltpu.async_copy(idxt_hbm.at[pl.ds(base_row, CHUNK)],
                                 idxt_rows, sem),
            ]
            for cp in cps:
                cp.wait()

            def body(g, carry, _ch=ch):
                mn, mx = carry
                e = g * LN + iota
                p = _div20(e)
                j = e - 20 * p
                c = plsc.load_gather(idxt_rows, [p, j])
                inv = c >= L
                vd = plsc.load_gather(dist_rows, [p, c])
                va = plsc.load_gather(angle_rows, [p, c])
                vd = jnp.where(inv, zero, vd)
                va = jnp.where(inv, zero, va)
                orow = _ch * CHUNK + p
                plsc.store_scatter(dist_o, [orow, j], vd)
                plsc.store_scatter(angle_o, [orow, j], va)
                return jnp.minimum(mn, vd), jnp.maximum(mx, vd)

            vmin, vmax = lax.fori_loop(0, cw // LN, body, (vmin, vmax))

        min_v[...] = vmin
        max_v[...] = vmax
        pltpu.sync_copy(dist_o, dist_g.at[pl.ds(wid * hb, hb)])
        pltpu.sync_copy(angle_o, angle_g.at[pl.ds(wid * hb, hb)])
        pltpu.sync_copy(min_v, mins.at[wid])
        pltpu.sync_copy(max_v, maxs.at[wid])

    return k(dist_r, angle_r, idxt_r, qvec2d)


def _tc_assemble(idx_tT, dist_gT, angle_gT, mins, maxs, h, w,
                 interpret=False):
    out_w = NCLS * w + 2 * w
    BH = 2048
    grid = (h // BH,)
    pt_ = dist_gT.shape[0]

    def body(idx_ref, dist_ref, angle_ref, mins_ref, maxs_ref, out_ref):
        gmin = jnp.min(mins_ref[...])
        gmax = jnp.max(maxs_ref[...])
        scale = 1.0 / (gmax - gmin)
        idxf = idx_ref[...].astype(jnp.bfloat16)
        qj = lax.broadcasted_iota(jnp.int32, (NCLS * w, w), 0) // NCLS
        jj = lax.broadcasted_iota(jnp.int32, (NCLS * w, w), 1)
        sel = (qj == jj).astype(jnp.bfloat16)
        rep = jnp.dot(sel, idxf, preferred_element_type=jnp.float32)
        cls = (lax.broadcasted_iota(jnp.int32, (NCLS * w, BH), 0)
               % NCLS).astype(jnp.float32)
        out_ref[:NCLS * w, :] = (rep == cls).astype(jnp.float32)
        out_ref[NCLS * w:NCLS * w + w, :] = (
            dist_ref[...][:w, :] - gmin) * scale
        out_ref[NCLS * w + w:, :] = angle_ref[...][:w, :]

    return pl.pallas_call(
        body,
        grid=grid,
        in_specs=[
            pl.BlockSpec((w, BH), lambda i: (0, i)),
            pl.BlockSpec((pt_, BH), lambda i: (0, i)),
            pl.BlockSpec((pt_, BH), lambda i: (0, i)),
            pl.BlockSpec(mins.shape, lambda i: (0, 0)),
            pl.BlockSpec(maxs.shape, lambda i: (0, 0)),
        ],
        out_specs=pl.BlockSpec((out_w, BH), lambda i: (0, i)),
        out_shape=jax.ShapeDtypeStruct((out_w, h), jnp.float32),
        interpret=interpret,
    )(idx_tT, dist_gT, angle_gT, mins, maxs)


def kernel(dist, angle, idx_t, index_t, index_h):
    N, L = dist.shape
    h, w = idx_t.shape
    dist_r = _tc_relayout(dist.T)
    angle_r = _tc_relayout(angle.T)
    idxt_r = _tc_relayout(index_t.T)
    qvec2d = index_h.astype(jnp.int32).reshape(h // 128, 128)
    dist_g, angle_g, mins, maxs = _sc_gather(dist_r, angle_r, idxt_r,
                                             qvec2d, h, w, L)
    x_t = _tc_assemble(idx_t.T, dist_g.T, angle_g.T, mins, maxs, h, w)
    return x_t.T


# speedup vs baseline: 2.0043x; 1.0787x over previous
import functools

import jax
import jax.numpy as jnp
from jax import lax
from jax.experimental import pallas as pl
from jax.experimental.pallas import tpu as pltpu
from jax.experimental.pallas import tpu_sc as plsc

NCLS = 22
CHUNK = 128
PITCH = 128


def _rup8(x):
    return ((x + 7) // 8) * 8


def _div20(x):
    return ((x >> 2) * 13108) >> 16


def _tc_relayout(tt):
    d, n = tt.shape
    BC = 2048
    nb = (n + BC - 1) // BC

    def body(in_ref, out_ref):
        out_ref[:, :d] = in_ref[...].T

    return pl.pallas_call(
        body,
        grid=(nb,),
        in_specs=[pl.BlockSpec((d, BC), lambda i: (0, i))],
        out_specs=pl.BlockSpec((BC, PITCH), lambda i: (i, 0)),
        out_shape=jax.ShapeDtypeStruct((nb * BC, PITCH), tt.dtype),
    )(tt)


def _sc_gather_one(tab_r, idxt_r, qvec2d, h, w, L, with_minmax):
    info = plsc.get_sparse_core_info()
    NC, NS, LN = info.num_cores, info.num_subcores, info.num_lanes
    NW = NC * NS
    hb = h // NW
    n_chunks = hb // CHUNK
    cw = CHUNK * w
    pt_ = _rup8(w)
    mesh = plsc.VectorSubcoreMesh(core_axis_name="c", subcore_axis_name="s")

    out_type = [jax.ShapeDtypeStruct((h, pt_), jnp.float32)]
    scratch = [
        pltpu.VMEM((1, 128), jnp.int32),
        pltpu.VMEM((CHUNK, PITCH), jnp.float32),
        pltpu.VMEM((CHUNK, PITCH), jnp.int32),
        pltpu.VMEM((hb, pt_), jnp.float32),
    ]
    if with_minmax:
        out_type += [jax.ShapeDtypeStruct((NW, LN), jnp.float32),
                     jax.ShapeDtypeStruct((NW, LN), jnp.float32)]
        scratch += [pltpu.VMEM((LN,), jnp.float32),
                    pltpu.VMEM((LN,), jnp.float32)]
    scratch.append(pltpu.SemaphoreType.DMA)

    @functools.partial(
        pl.kernel,
        out_type=tuple(out_type),
        mesh=mesh,
        compiler_params=pltpu.CompilerParams(needs_layout_passes=False,
                                             use_tc_tiling_on_sc=False),
        scratch_types=tuple(scratch),
    )
    def k(tab_hbm, idxt_hbm, qvec_hbm, *rest):
        if with_minmax:
            (tab_g, mins, maxs,
             idx_s, tab_rows, idxt_rows, tab_o, min_v, max_v, sem) = rest
        else:
            tab_g, idx_s, tab_rows, idxt_rows, tab_o, sem = rest
        wid = lax.axis_index("s") * NC + lax.axis_index("c")
        inf = jnp.full((LN,), jnp.inf, dtype=jnp.float32)
        zero = jnp.zeros((LN,), dtype=jnp.float32)
        iota = lax.broadcasted_iota(jnp.int32, (LN,), 0)
        vmin, vmax = inf, -inf
        for ch in range(n_chunks):
            base_row = wid * hb + ch * CHUNK
            pltpu.sync_copy(qvec_hbm.at[pl.ds(base_row // 128, 1)], idx_s)
            cps = [
                pltpu.async_copy(tab_hbm.at[idx_s.at[0]], tab_rows, sem),
                pltpu.async_copy(idxt_hbm.at[pl.ds(base_row, CHUNK)],
                                 idxt_rows, sem),
            ]
            for cp in cps:
                cp.wait()

            def body(g, carry, _ch=ch):
                mn, mx = carry
                e = g * LN + iota
                p = _div20(e)
                j = e - 20 * p
                c = plsc.load_gather(idxt_rows, [p, j])
                inv = c >= L
                v = plsc.load_gather(tab_rows, [p, c])
                v = jnp.where(inv, zero, v)
                plsc.store_scatter(tab_o, [_ch * CHUNK + p, j], v)
                if with_minmax:
                    return jnp.minimum(mn, v), jnp.maximum(mx, v)
                return mn, mx

            vmin, vmax = lax.fori_loop(0, cw // LN, body, (vmin, vmax))

        pltpu.sync_copy(tab_o, tab_g.at[pl.ds(wid * hb, hb)])
        if with_minmax:
            min_v[...] = vmin
            max_v[...] = vmax
            pltpu.sync_copy(min_v, mins.at[wid])
            pltpu.sync_copy(max_v, maxs.at[wid])

    return k(tab_r, idxt_r, qvec2d)


def _tc_assemble(idx_tT, dist_gT, angle_gT, mins, maxs, h, w,
                 interpret=False):
    out_w = NCLS * w + 2 * w
    BH = 2048
    grid = (h // BH,)
    pt_ = dist_gT.shape[0]

    def body(idx_ref, dist_ref, angle_ref, mins_ref, maxs_ref, out_ref):
        gmin = jnp.min(mins_ref[...])
        gmax = jnp.max(maxs_ref[...])
        scale = 1.0 / (gmax - gmin)
        idxf = idx_ref[...].astype(jnp.bfloat16)
        qj = lax.broadcasted_iota(jnp.int32, (NCLS * w, w), 0) // NCLS
        jj = lax.broadcasted_iota(jnp.int32, (NCLS * w, w), 1)
        sel = (qj == jj).astype(jnp.bfloat16)
        rep = jnp.dot(sel, idxf, preferred_element_type=jnp.float32)
        cls = (lax.broadcasted_iota(jnp.int32, (NCLS * w, BH), 0)
               % NCLS).astype(jnp.float32)
        out_ref[:NCLS * w, :] = (rep == cls).astype(jnp.float32)
        out_ref[NCLS * w:NCLS * w + w, :] = (
            dist_ref[...][:w, :] - gmin) * scale
        out_ref[NCLS * w + w:, :] = angle_ref[...][:w, :]

    return pl.pallas_call(
        body,
        grid=grid,
        in_specs=[
            pl.BlockSpec((w, BH), lambda i: (0, i)),
            pl.BlockSpec((pt_, BH), lambda i: (0, i)),
            pl.BlockSpec((pt_, BH), lambda i: (0, i)),
            pl.BlockSpec(mins.shape, lambda i: (0, 0)),
            pl.BlockSpec(maxs.shape, lambda i: (0, 0)),
        ],
        out_specs=pl.BlockSpec((out_w, BH), lambda i: (0, i)),
        out_shape=jax.ShapeDtypeStruct((out_w, h), jnp.float32),
        interpret=interpret,
    )(idx_tT, dist_gT, angle_gT, mins, maxs)


def kernel(dist, angle, idx_t, index_t, index_h):
    N, L = dist.shape
    h, w = idx_t.shape
    qvec2d = index_h.astype(jnp.int32).reshape(h // 128, 128)
    idxt_r = _tc_relayout(index_t.T)
    dist_r = _tc_relayout(dist.T)
    dist_g, mins, maxs = _sc_gather_one(dist_r, idxt_r, qvec2d, h, w, L,
                                        True)
    angle_r = _tc_relayout(angle.T)
    (angle_g,) = _sc_gather_one(angle_r, idxt_r, qvec2d, h, w, L, False)
    x_t = _tc_assemble(idx_t.T, dist_g.T, angle_g.T, mins, maxs, h, w)
    return x_t.T
